# baseline (speedup 1.0000x reference)
import functools

import jax
import jax.numpy as jnp
from jax import lax
from jax.experimental import pallas as pl
from jax.experimental.pallas import tpu as pltpu
from jax.experimental.pallas import tpu_sc as plsc

NF = 26
VOCAB = 1000
ED = 64
NCONT = 13
BATCH = 4096
EPS = 1e-5

NC, NS, L = 2, 16, 16
NW = NC * NS
ROWS_W = BATCH // NW
IDX_W = ROWS_W * NF
STEP = 128
STEPS = IDX_W // STEP

BT = 256
D_FEAT = NF * ED


def _gather_body(tab_hbm, idx_hbm, out_hbm, idxv, rows, sem):
    wid = lax.axis_index("s") * NC + lax.axis_index("c")
    pltpu.sync_copy(idx_hbm.at[wid], idxv)
    for i in range(STEPS * (STEP // L)):
        j, c = divmod(i, STEP // L)
        c *= L
        pos = lax.iota(jnp.int32, L) + (j * STEP + c)
        off = (pos % NF) * VOCAB
        idxv[j, pl.ds(c, L)] = idxv[j, pl.ds(c, L)] + off
    cps = [
        pltpu.async_copy(
            tab_hbm.at[idxv.at[t]],
            rows.at[pl.ds(t * STEP, STEP)],
            sem,
        )
        for t in range(STEPS)
    ]
    for cp in cps:
        cp.wait()
    pltpu.sync_copy(rows, out_hbm.at[pl.ds(wid * IDX_W, IDX_W)])


def _sc_gather(tab, idx3):
    mesh = plsc.VectorSubcoreMesh(
        core_axis_name="c", subcore_axis_name="s", num_cores=NC, num_subcores=NS
    )
    f = pl.kernel(
        _gather_body,
        out_type=jax.ShapeDtypeStruct((BATCH * NF, ED), jnp.bfloat16),
        mesh=mesh,
        scratch_types=[
            pltpu.VMEM((STEPS, STEP), jnp.int32),
            pltpu.VMEM((STEPS * STEP, ED), jnp.bfloat16),
            pltpu.SemaphoreType.DMA,
        ],
        compiler_params=pltpu.CompilerParams(use_tc_tiling_on_sc=False),
    )
    return f(tab, idx3)


def _mlp_body(
    xf, xc, g0, be0, w1f, w1c, b1, g1, be1, w2, b2, g2, be2, w3, b3, g3, be3,
    wout, bout, out
):
    inv = 1.0 / jnp.sqrt(jnp.float32(1.0) + EPS)
    dot = functools.partial(lax.dot_general, preferred_element_type=jnp.float32)
    ct = (((1,), (1,)), ((), ()))
    xcb = ((xc[...] * inv) * g0[...] + be0[...]).astype(jnp.bfloat16)
    h = dot(xf[...], w1f[...], ct) + dot(xcb, w1c[...], ct)
    h = jnp.maximum(h + b1[...], 0.0)
    h = ((h * inv) * g1[...] + be1[...]).astype(jnp.bfloat16)
    h = jnp.maximum(dot(h, w2[...], ct) + b2[...], 0.0)
    h = ((h * inv) * g2[...] + be2[...]).astype(jnp.bfloat16)
    h = jnp.maximum(dot(h, w3[...], ct) + b3[...], 0.0)
    h = ((h * inv) * g3[...] + be3[...]).astype(jnp.bfloat16)
    out[...] = dot(wout[...], h, ct) + bout[0]


def _row(v):
    return v.reshape(1, -1)


def _full_spec(a):
    return pl.BlockSpec(a.shape, lambda i: (0, 0))


def kernel(x_categories_tensor101, x_continuous_tensor101, emb_tables, bn0_gamma,
           bn0_beta, W1, b1, g1, be1, W2, b2, g2, be2, W3, b3, g3, be3, Wout, bout):
    bf = jnp.bfloat16
    tab = emb_tables.reshape(NF * VOCAB, ED).astype(bf)
    idx3 = x_categories_tensor101.astype(jnp.int32).reshape(NW, STEPS, STEP)
    xf = _sc_gather(tab, idx3).reshape(BATCH, D_FEAT)
    xc = x_continuous_tensor101
    params = [
        _row(bn0_gamma), _row(bn0_beta),
        W1[:, :D_FEAT].astype(bf), W1[:, D_FEAT:].astype(bf),
        _row(b1), _row(g1), _row(be1),
        W2.astype(bf), _row(b2), _row(g2), _row(be2),
        W3.astype(bf), _row(b3), _row(g3), _row(be3),
        Wout.astype(bf),
    ]
    out = pl.pallas_call(
        _mlp_body,
        grid=(BATCH // BT,),
        in_specs=[
            pl.BlockSpec((BT, D_FEAT), lambda i: (i, 0)),
            pl.BlockSpec((BT, NCONT), lambda i: (i, 0)),
        ] + [_full_spec(p) for p in params]
          + [pl.BlockSpec(memory_space=pltpu.SMEM)],
        out_specs=pl.BlockSpec((1, BT), lambda i: (0, i)),
        out_shape=jax.ShapeDtypeStruct((1, BATCH), jnp.float32),
    )(xf, xc, *params, bout)
    return out.reshape(BATCH, 1)


# speedup vs baseline: 1.2869x; 1.2869x over previous
import functools

import jax
import jax.numpy as jnp
from jax import lax
from jax.experimental import pallas as pl
from jax.experimental.pallas import tpu as pltpu
from jax.experimental.pallas import tpu_sc as plsc

NF = 26
VOCAB = 1000
ED = 64
NCONT = 13
BATCH = 4096
EPS = 1e-5

NC, NS, L = 2, 16, 16
NW = NC * NS
ROWS_W = BATCH // NW
IDX_W = ROWS_W * NF
STEP = 128
STEPS = IDX_W // STEP
HALF = STEPS // 2

BT = 256
D_FEAT = NF * ED


def _gather_body(tab_hbm, idx_hbm, out_hbm, idxv, rows, sem):
    wid = lax.axis_index("s") * NC + lax.axis_index("c")
    pltpu.sync_copy(idx_hbm.at[wid], idxv)
    for i in range(STEPS * (STEP // L)):
        j, c = divmod(i, STEP // L)
        c *= L
        pos = lax.iota(jnp.int32, L) + (j * STEP + c)
        off = (pos % NF) * VOCAB
        idxv[j, pl.ds(c, L)] = idxv[j, pl.ds(c, L)] + off
    base = wid * IDX_W
    for h in range(2):
        cps = [
            pltpu.async_copy(
                tab_hbm.at[idxv.at[h * HALF + t]],
                rows.at[pl.ds(t * STEP, STEP)],
                sem,
            )
            for t in range(HALF)
        ]
        for cp in cps:
            cp.wait()
        pltpu.sync_copy(rows, out_hbm.at[pl.ds(base + h * HALF * STEP, HALF * STEP)])


def _sc_gather(tab, idx3):
    mesh = plsc.VectorSubcoreMesh(
        core_axis_name="c", subcore_axis_name="s", num_cores=NC, num_subcores=NS
    )
    f = pl.kernel(
        _gather_body,
        out_type=jax.ShapeDtypeStruct((BATCH * NF, ED), jnp.float32),
        mesh=mesh,
        scratch_types=[
            pltpu.VMEM((STEPS, STEP), jnp.int32),
            pltpu.VMEM((HALF * STEP, ED), jnp.float32),
            pltpu.SemaphoreType.DMA,
        ],
        compiler_params=pltpu.CompilerParams(use_tc_tiling_on_sc=False),
    )
    return f(tab, idx3)


def _mlp_body(
    xf, xc, g0, be0, w1f, w1c, b1, g1, be1, w2, b2, g2, be2, w3, b3, g3, be3,
    wout, bout, out
):
    inv = 1.0 / jnp.sqrt(jnp.float32(1.0) + EPS)
    dot = functools.partial(lax.dot_general, preferred_element_type=jnp.float32)
    ct = (((1,), (1,)), ((), ()))
    xcb = ((xc[...] * inv) * g0[...] + be0[...]).astype(jnp.bfloat16)
    h = dot(xf[...].astype(jnp.bfloat16), w1f[...], ct) + dot(xcb, w1c[...], ct)
    h = jnp.maximum(h + b1[...], 0.0)
    h = ((h * inv) * g1[...] + be1[...]).astype(jnp.bfloat16)
    h = jnp.maximum(dot(h, w2[...], ct) + b2[...], 0.0)
    h = ((h * inv) * g2[...] + be2[...]).astype(jnp.bfloat16)
    h = jnp.maximum(dot(h, w3[...], ct) + b3[...], 0.0)
    h = ((h * inv) * g3[...] + be3[...]).astype(jnp.bfloat16)
    out[...] = dot(wout[...], h, ct) + bout[0]


def _row(v):
    return v.reshape(1, -1)


def _full_spec(a):
    return pl.BlockSpec(a.shape, lambda i: (0, 0))


def kernel(x_categories_tensor101, x_continuous_tensor101, emb_tables, bn0_gamma,
           bn0_beta, W1, b1, g1, be1, W2, b2, g2, be2, W3, b3, g3, be3, Wout, bout):
    bf = jnp.bfloat16
    tab = emb_tables.reshape(NF * VOCAB, ED)
    idx3 = x_categories_tensor101.astype(jnp.int32).reshape(NW, STEPS, STEP)
    xf = _sc_gather(tab, idx3).reshape(BATCH, D_FEAT)
    xc = x_continuous_tensor101
    params = [
        _row(bn0_gamma), _row(bn0_beta),
        W1[:, :D_FEAT].astype(bf), W1[:, D_FEAT:].astype(bf),
        _row(b1), _row(g1), _row(be1),
        W2.astype(bf), _row(b2), _row(g2), _row(be2),
        W3.astype(bf), _row(b3), _row(g3), _row(be3),
        Wout.astype(bf),
    ]
    out = pl.pallas_call(
        _mlp_body,
        grid=(BATCH // BT,),
        in_specs=[
            pl.BlockSpec((BT, D_FEAT), lambda i: (i, 0)),
            pl.BlockSpec((BT, NCONT), lambda i: (i, 0)),
        ] + [_full_spec(p) for p in params]
          + [pl.BlockSpec(memory_space=pltpu.SMEM)],
        out_specs=pl.BlockSpec((1, BT), lambda i: (0, i)),
        out_shape=jax.ShapeDtypeStruct((1, BATCH), jnp.float32),
    )(xf, xc, *params, bout)
    return out.reshape(BATCH, 1)
